# baseline (speedup 1.0000x reference)
import functools

import jax
import jax.numpy as jnp
from jax import lax
from jax.experimental import pallas as pl
from jax.experimental.pallas import tpu as pltpu
from jax.experimental.pallas import tpu_sc as plsc

B = 16384
EMB_DIM = 32
PACK = 4
PD = PACK * EMB_DIM

_info = plsc.get_sparse_core_info()
_NC, _NS = _info.num_cores, _info.num_subcores
_NW = _NC * _NS
_BPW = B // _NW
_CH = _BPW // 2


def _make_gather(nu_rows, nf_rows):
    mesh = plsc.VectorSubcoreMesh(core_axis_name="c", subcore_axis_name="s")

    @functools.partial(
        pl.kernel,
        mesh=mesh,
        out_type=(
            jax.ShapeDtypeStruct((B, PD), jnp.float32),
            jax.ShapeDtypeStruct((B, PD), jnp.float32),
        ),
        scratch_types=[
            pltpu.VMEM((_CH,), jnp.int32),
            pltpu.VMEM((_CH,), jnp.int32),
            pltpu.VMEM((_CH, PD), jnp.float32),
            pltpu.VMEM((_CH, PD), jnp.float32),
            pltpu.SemaphoreType.DMA,
            pltpu.SemaphoreType.DMA,
        ],
    )
    def gather_k(ut_hbm, uid_hbm, ft_hbm, fid_hbm, out_u, out_f,
                 idx0, idx1, rows0, rows1, sem0, sem1):
        wid = lax.axis_index("s") * _NC + lax.axis_index("c")
        base = wid * _BPW
        pltpu.sync_copy(uid_hbm.at[pl.ds(base, _CH)], idx0)
        pltpu.sync_copy(uid_hbm.at[pl.ds(base + _CH, _CH)], idx1)
        c0 = pltpu.async_copy(ut_hbm.at[idx0], rows0, sem0)
        c1 = pltpu.async_copy(ut_hbm.at[idx1], rows1, sem1)
        c0.wait()
        pltpu.sync_copy(rows0, out_u.at[pl.ds(base, _CH)])
        c1.wait()
        pltpu.sync_copy(rows1, out_u.at[pl.ds(base + _CH, _CH)])
        pltpu.sync_copy(fid_hbm.at[pl.ds(base, _CH)], idx0)
        pltpu.sync_copy(fid_hbm.at[pl.ds(base + _CH, _CH)], idx1)
        c0 = pltpu.async_copy(ft_hbm.at[idx0], rows0, sem0)
        c1 = pltpu.async_copy(ft_hbm.at[idx1], rows1, sem1)
        c0.wait()
        pltpu.sync_copy(rows0, out_f.at[pl.ds(base, _CH)])
        c1.wait()
        pltpu.sync_copy(rows1, out_f.at[pl.ds(base + _CH, _CH)])

    return gather_k


_BM = 1024


def _mlp_body(xu4_ref, xf4_ref, mu_ref, mf_ref,
              w1_ref, b1_ref, w2_ref, b2_ref, w3_ref, b3_ref, out_ref):
    xu4 = xu4_ref[...]
    xf4 = xf4_ref[...]
    i = pl.program_id(0)
    nb = pl.num_programs(0)
    onehot = (lax.broadcasted_iota(jnp.int32, (nb, 1), 0) == i).astype(
        jnp.float32)
    mu = jnp.dot(mu_ref[...].astype(jnp.float32), onehot,
                 preferred_element_type=jnp.float32)
    mf = jnp.dot(mf_ref[...].astype(jnp.float32), onehot,
                 preferred_element_type=jnp.float32)
    xu = xu4[:, 0:EMB_DIM]
    xf = xf4[:, 0:EMB_DIM]
    for c in range(1, PACK):
        s = slice(c * EMB_DIM, (c + 1) * EMB_DIM)
        xu = jnp.where(mu == c, xu4[:, s], xu)
        xf = jnp.where(mf == c, xf4[:, s], xf)
    x = jnp.concatenate([xu, xf], axis=1)
    h = jnp.dot(x, w1_ref[...], preferred_element_type=jnp.float32)
    h = jnp.maximum(h + b1_ref[...], 0.0)
    h = jnp.dot(h, w2_ref[...], preferred_element_type=jnp.float32)
    h = jnp.maximum(h + b2_ref[...], 0.0)
    out_ref[...] = (
        jnp.dot(h, w3_ref[...], preferred_element_type=jnp.float32)
        + b3_ref[...]
    )


def _mlp(xu4, xf4, mu_col, mf_col, W1, b1, W2, b2, W3, b3):
    d1, d2, d3 = W1.shape[1], W2.shape[1], W3.shape[1]
    grid = (B // _BM,)

    def full(shape):
        return pl.BlockSpec(shape, lambda i: (0, 0))

    return pl.pallas_call(
        _mlp_body,
        grid=grid,
        in_specs=[
            pl.BlockSpec((_BM, PD), lambda i: (i, 0)),
            pl.BlockSpec((_BM, PD), lambda i: (i, 0)),
            pl.BlockSpec((_BM, B // _BM), lambda i: (0, 0)),
            pl.BlockSpec((_BM, B // _BM), lambda i: (0, 0)),
            full(W1.shape),
            full((1, d1)),
            full(W2.shape),
            full((1, d2)),
            full(W3.shape),
            full((1, d3)),
        ],
        out_specs=pl.BlockSpec((_BM, d3), lambda i: (i, 0)),
        out_shape=jax.ShapeDtypeStruct((B, d3), jnp.float32),
    )(xu4, xf4, mu_col, mf_col, W1, b1.reshape(1, d1), W2,
      b2.reshape(1, d2), W3, b3.reshape(1, d3))


def _pack_table(table):
    v, d = table.shape
    pad = (-v) % PACK
    t = jnp.concatenate([table, jnp.zeros((pad, d), table.dtype)], axis=0)
    return t.reshape((v + pad) // PACK, PACK * d)


def kernel(user_ids, emotion_ids, user_table, feeling_table,
           W1, b1, W2, b2, W3, b3):
    uid = user_ids.astype(jnp.int32)
    fid = emotion_ids.astype(jnp.int32)
    ut_g = _pack_table(user_table)
    ft_g = _pack_table(feeling_table)
    uq, um = uid // PACK, uid % PACK
    fq, fm = fid // PACK, fid % PACK
    mu_col = um.reshape(B // _BM, _BM).T
    mf_col = fm.reshape(B // _BM, _BM).T
    gather = _make_gather(ut_g.shape[0], ft_g.shape[0])
    xu4, xf4 = gather(ut_g, uq, ft_g, fq)
    return _mlp(xu4, xf4, mu_col, mf_col, W1, b1, W2, b2, W3, b3)
